# baseline (speedup 1.0000x reference)
import functools

import jax
import jax.numpy as jnp
from jax import lax
from jax.experimental import pallas as pl
from jax.experimental.pallas import tpu as pltpu
from jax.experimental.pallas import tpu_sc as plsc

NUM_EMB = 1000000
DIM = 64
B_TOK = 16384
SEQ = 50
B = B_TOK * SEQ

NC = 2
NS = 16
NW = NC * NS
B_PER_W = B // NW
CHUNK = 200
NCHUNK = B_PER_W // CHUNK
NB = 4
NPAIR = NCHUNK // (2 * NB)
assert NCHUNK == NPAIR * 2 * NB


@functools.partial(
    pl.kernel,
    out_type=jax.ShapeDtypeStruct((B, DIM), jnp.float32),
    mesh=plsc.VectorSubcoreMesh(core_axis_name="c", subcore_axis_name="s"),
    scratch_types=[
        pltpu.VMEM((NCHUNK, CHUNK), jnp.int32),
        [pltpu.VMEM((CHUNK, DIM), jnp.float32) for _ in range(2 * NB)],
        [pltpu.SemaphoreType.DMA for _ in range(2 * NB)],
    ],
    compiler_params=pltpu.CompilerParams(use_tc_tiling_on_sc=False),
)
def _gather_kernel(idx_hbm, table_hbm, out_hbm, idx_v, bufs, sems):
    bufs_a, bufs_b = bufs[:NB], bufs[NB:]
    sem_a, sem_b = sems[:NB], sems[NB:]
    wid = lax.axis_index("s") * NC + lax.axis_index("c")
    base = wid * B_PER_W
    pltpu.sync_copy(idx_hbm.at[wid], idx_v)

    def out_slice(c):
        return out_hbm.at[pl.ds(base + c * CHUNK, CHUNK)]

    for b in range(NB):
        pltpu.async_copy(table_hbm.at[idx_v.at[b]], bufs_a[b], sem_a[b])

    def pair(p):
        g0 = p * 2 * NB
        for b in range(NB):
            c = g0 + b
            pltpu.make_async_copy(table_hbm.at[idx_v.at[c]], bufs_a[b], sem_a[b]).wait()
            pltpu.async_copy(bufs_a[b], out_slice(c), sem_a[b])
        for b in range(NB):
            c = g0 + NB + b

            @pl.when(p > 0)
            def _():
                pltpu.make_async_copy(bufs_b[b], out_slice(0), sem_b[b]).wait()

            pltpu.async_copy(table_hbm.at[idx_v.at[c]], bufs_b[b], sem_b[b])
        for b in range(NB):
            c = g0 + NB + b
            pltpu.make_async_copy(table_hbm.at[idx_v.at[c]], bufs_b[b], sem_b[b]).wait()
            pltpu.async_copy(bufs_b[b], out_slice(c), sem_b[b])
        for b in range(NB):
            pltpu.make_async_copy(bufs_a[b], out_slice(0), sem_a[b]).wait()

            @pl.when(p < NPAIR - 1)
            def _():
                pltpu.async_copy(
                    table_hbm.at[idx_v.at[g0 + 2 * NB + b]], bufs_a[b], sem_a[b]
                )

    pl.loop(0, NPAIR)(pair)

    for b in range(NB):
        pltpu.make_async_copy(bufs_b[b], out_slice(0), sem_b[b]).wait()


def kernel(token_ids, weights):
    idx = token_ids.reshape(NW, NCHUNK, CHUNK).astype(jnp.int32)
    out = _gather_kernel(idx, weights)
    return out.reshape(B_TOK, SEQ, DIM)


# speedup vs baseline: 1.0004x; 1.0004x over previous
import functools

import jax
import jax.numpy as jnp
from jax import lax
from jax.experimental import pallas as pl
from jax.experimental.pallas import tpu as pltpu
from jax.experimental.pallas import tpu_sc as plsc

NUM_EMB = 1000000
DIM = 64
B_TOK = 16384
SEQ = 50
B = B_TOK * SEQ

NC = 2
NS = 16
NW = NC * NS
B_PER_W = B // NW
CHUNK = 128
NCHUNK = B_PER_W // CHUNK
NB = 5
NPAIR = NCHUNK // (2 * NB)
assert NCHUNK == NPAIR * 2 * NB


@functools.partial(
    pl.kernel,
    out_type=jax.ShapeDtypeStruct((B, DIM), jnp.float32),
    mesh=plsc.VectorSubcoreMesh(core_axis_name="c", subcore_axis_name="s"),
    scratch_types=[
        pltpu.VMEM((NCHUNK, CHUNK), jnp.int32),
        [pltpu.VMEM((CHUNK, DIM), jnp.float32) for _ in range(2 * NB)],
        [pltpu.SemaphoreType.DMA for _ in range(2 * NB)],
    ],
    compiler_params=pltpu.CompilerParams(use_tc_tiling_on_sc=False),
)
def _gather_kernel(idx_hbm, table_hbm, out_hbm, idx_v, bufs, sems):
    bufs_a, bufs_b = bufs[:NB], bufs[NB:]
    sem_a, sem_b = sems[:NB], sems[NB:]
    wid = lax.axis_index("s") * NC + lax.axis_index("c")
    base = wid * B_PER_W
    pltpu.sync_copy(idx_hbm.at[wid], idx_v)

    def out_slice(c):
        return out_hbm.at[pl.ds(base + c * CHUNK, CHUNK)]

    for b in range(NB):
        pltpu.async_copy(table_hbm.at[idx_v.at[b]], bufs_a[b], sem_a[b])

    def pair(p):
        g0 = p * 2 * NB
        for b in range(NB):
            c = g0 + b
            pltpu.make_async_copy(table_hbm.at[idx_v.at[c]], bufs_a[b], sem_a[b]).wait()
            pltpu.async_copy(bufs_a[b], out_slice(c), sem_a[b])
        for b in range(NB):
            c = g0 + NB + b

            @pl.when(p > 0)
            def _():
                pltpu.make_async_copy(bufs_b[b], out_slice(0), sem_b[b]).wait()

            pltpu.async_copy(table_hbm.at[idx_v.at[c]], bufs_b[b], sem_b[b])
        for b in range(NB):
            c = g0 + NB + b
            pltpu.make_async_copy(table_hbm.at[idx_v.at[c]], bufs_b[b], sem_b[b]).wait()
            pltpu.async_copy(bufs_b[b], out_slice(c), sem_b[b])
        for b in range(NB):
            pltpu.make_async_copy(bufs_a[b], out_slice(0), sem_a[b]).wait()

            @pl.when(p < NPAIR - 1)
            def _():
                pltpu.async_copy(
                    table_hbm.at[idx_v.at[g0 + 2 * NB + b]], bufs_a[b], sem_a[b]
                )

    pl.loop(0, NPAIR)(pair)

    for b in range(NB):
        pltpu.make_async_copy(bufs_b[b], out_slice(0), sem_b[b]).wait()


def kernel(token_ids, weights):
    idx = token_ids.reshape(NW, NCHUNK, CHUNK).astype(jnp.int32)
    out = _gather_kernel(idx, weights)
    return out.reshape(B_TOK, SEQ, DIM)
